# baseline (speedup 1.0000x reference)
import jax
import jax.numpy as jnp
from jax import lax
from jax.experimental import pallas as pl
from jax.experimental.pallas import tpu as pltpu
from jax.experimental.pallas import tpu_sc as plsc

NUM_SEQ = 4096
NUM_CLS = 1024
BLANK = 0
_BIG = 1 << 30
_NEG_INF = float("-inf")

_SC_ROWS = 2048
_TC_ROWS = NUM_SEQ - _SC_ROWS
_S = _TC_ROWS

_NC = 2
_NS = 16
_NW = _NC * _NS
_ROWS_PER_W = _SC_ROWS // _NW
_BLK = 16
_NBLK = _ROWS_PER_W // _BLK
_NBUF = min(4, _NBLK)
_RIF = 8
_NCHUNK = NUM_CLS // 16

_TC_R = 512


def _take16(x, idx):
    dn = lax.GatherDimensionNumbers(
        offset_dims=(), collapsed_slice_dims=(0,), start_index_map=(0,))
    return lax.gather(x, idx[:, None], dn, slice_sizes=(1,),
                      mode=lax.GatherScatterMode.PROMISE_IN_BOUNDS)


def _scan_rows(rows, iota):
    init = tuple(
        (jnp.full((16,), _NEG_INF, jnp.float32), jnp.zeros((16,), jnp.int32))
        for _ in rows
    )

    def body(j, accs):
        jb = j * 16
        jvec = jnp.full((16,), jb, jnp.int32)
        out = []
        for r, (buf, row) in enumerate(rows):
            m, ix = accs[r]
            v = buf[row, pl.ds(jb, 16)]
            upd = v > m
            out.append((jnp.where(upd, v, m), jnp.where(upd, jvec, ix)))
        return tuple(out)

    accs = lax.fori_loop(0, _NCHUNK, body, init)

    results = []
    for m, ix in accs:
        mx = jnp.max(m)
        cand = jnp.where(m == mx, ix + iota, _BIG)
        results.append(jnp.min(cand))
    return results


def _sc_body(em_hbm, out_hbm, b0, b1, b2, b3, pbuf, outv, s0, s1, s2, s3):
    bufs = (b0, b1, b2, b3)
    sems = (s0, s1, s2, s3)
    cid = lax.axis_index("c")
    sid = lax.axis_index("s")
    wid = sid * _NC + cid
    row0 = _S + wid * _ROWS_PER_W

    def start(b):
        return pltpu.async_copy(
            em_hbm.at[pl.ds(row0 + b * _BLK, _BLK), :],
            bufs[b % _NBUF], sems[b % _NBUF])

    handles = [start(b) for b in range(_NBUF)]

    riota = lax.iota(jnp.int32, 16)
    shift_idx = jnp.maximum(riota - 1, 0)

    pcopy = pltpu.async_copy(em_hbm.at[pl.ds(row0 - 1, 1), :], pbuf, sems[0])

    prev_last = jnp.int32(-1)
    for b in range(_NBLK):
        if b == 0:
            pcopy.wait()
        handles[b % _NBUF].wait()
        buf = bufs[b % _NBUF]
        cs = []
        for r0 in range(0, _BLK, _RIF):
            rows = [(buf, r0 + r) for r in range(_RIF)]
            if b == 0 and r0 == 0:
                rows.append((pbuf, 0))
                res = _scan_rows(rows, riota)
                prev_last = res[_RIF]
                cs.extend(res[:_RIF])
            else:
                cs.extend(_scan_rows(rows, riota))
        idxv = jnp.full((16,), -1, jnp.int32)
        for r, c in enumerate(cs):
            idxv = jnp.where(riota == r, c, idxv)
        shifted = _take16(idxv, shift_idx)
        prevv = jnp.where(riota == 0, prev_last, shifted)
        keep = (idxv != prevv) & (idxv != BLANK)
        outv[pl.ds(b * _BLK, _BLK)] = jnp.where(keep, idxv, -1)
        prev_last = cs[_BLK - 1]
        if b + _NBUF < _NBLK:
            handles[b % _NBUF] = start(b + _NBUF)

    pltpu.sync_copy(outv, out_hbm.at[pl.ds(wid * _ROWS_PER_W, _ROWS_PER_W)])


_sc_decode = pl.kernel(
    _sc_body,
    out_type=jax.ShapeDtypeStruct((_SC_ROWS,), jnp.int32),
    mesh=plsc.VectorSubcoreMesh(core_axis_name="c", subcore_axis_name="s"),
    compiler_params=pltpu.CompilerParams(
        needs_layout_passes=False, disable_bounds_checks=True),
    scratch_types=[
        pltpu.VMEM((_BLK, NUM_CLS), jnp.float32),
        pltpu.VMEM((_BLK, NUM_CLS), jnp.float32),
        pltpu.VMEM((_BLK, NUM_CLS), jnp.float32),
        pltpu.VMEM((_BLK, NUM_CLS), jnp.float32),
        pltpu.VMEM((1, NUM_CLS), jnp.float32),
        pltpu.VMEM((_ROWS_PER_W,), jnp.int32),
        pltpu.SemaphoreType.DMA,
        pltpu.SemaphoreType.DMA,
        pltpu.SemaphoreType.DMA,
        pltpu.SemaphoreType.DMA,
    ],
)


def _tc_body(em_ref, out_ref, prev_ref):
    @pl.when(pl.program_id(0) == 0)
    def _init():
        prev_ref[0] = jnp.int32(-1)

    x = em_ref[...]
    m = jnp.max(x, axis=1, keepdims=True)
    colio = lax.broadcasted_iota(jnp.int32, x.shape, 1)
    cand = jnp.where(x == m, colio, _BIG)
    idx = jnp.min(cand, axis=1)
    ps = prev_ref[0]
    shifted = jnp.concatenate([jnp.full((1,), ps, jnp.int32), idx[:-1]])
    keep = (idx != shifted) & (idx != BLANK)
    out_ref[...] = jnp.where(keep, idx, -1)
    prev_ref[0] = idx[_TC_R - 1]


_tc_decode = pl.pallas_call(
    _tc_body,
    grid=(_TC_ROWS // _TC_R,),
    in_specs=[pl.BlockSpec((_TC_R, NUM_CLS), lambda i: (i, 0))],
    out_specs=pl.BlockSpec((_TC_R,), lambda i: (i,)),
    out_shape=jax.ShapeDtypeStruct((_TC_ROWS,), jnp.int32),
    scratch_shapes=[pltpu.SMEM((1,), jnp.int32)],
)


@jax.jit
def kernel(emission):
    sc_out = _sc_decode(emission)
    tc_out = _tc_decode(emission)
    return jnp.concatenate([tc_out, sc_out])


# speedup vs baseline: 1.0536x; 1.0536x over previous
import jax
import jax.numpy as jnp
from jax import lax
from jax.experimental import pallas as pl
from jax.experimental.pallas import tpu as pltpu
from jax.experimental.pallas import tpu_sc as plsc

NUM_SEQ = 4096
NUM_CLS = 1024
BLANK = 0
_BIG = 1 << 30
_NEG_INF = float("-inf")

_SC_ROWS = 1536
_TC_ROWS = NUM_SEQ - _SC_ROWS
_S = _TC_ROWS

_NC = 2
_NS = 16
_NW = _NC * _NS
_ROWS_PER_W = _SC_ROWS // _NW
_BLK = 16
_NBLK = _ROWS_PER_W // _BLK
_NBUF = min(4, _NBLK)
_RIF = 8
_NCHUNK = NUM_CLS // 16

_TC_R = 512


def _take16(x, idx):
    dn = lax.GatherDimensionNumbers(
        offset_dims=(), collapsed_slice_dims=(0,), start_index_map=(0,))
    return lax.gather(x, idx[:, None], dn, slice_sizes=(1,),
                      mode=lax.GatherScatterMode.PROMISE_IN_BOUNDS)


def _scan_rows(rows, iota):
    init = tuple(
        (jnp.full((16,), _NEG_INF, jnp.float32), jnp.zeros((16,), jnp.int32))
        for _ in rows
    )

    def body(j, accs):
        jb = j * 16
        jvec = jnp.full((16,), jb, jnp.int32)
        out = []
        for r, (buf, row) in enumerate(rows):
            m, ix = accs[r]
            v = buf[row, pl.ds(jb, 16)]
            upd = v > m
            out.append((jnp.where(upd, v, m), jnp.where(upd, jvec, ix)))
        return tuple(out)

    accs = lax.fori_loop(0, _NCHUNK, body, init)

    results = []
    for m, ix in accs:
        mx = jnp.max(m)
        cand = jnp.where(m == mx, ix + iota, _BIG)
        results.append(jnp.min(cand))
    return results


def _sc_body(em_hbm, out_hbm, b0, b1, b2, b3, pbuf, outv, s0, s1, s2, s3):
    bufs = (b0, b1, b2, b3)
    sems = (s0, s1, s2, s3)
    cid = lax.axis_index("c")
    sid = lax.axis_index("s")
    wid = sid * _NC + cid
    row0 = _S + wid * _ROWS_PER_W

    def start(b):
        return pltpu.async_copy(
            em_hbm.at[pl.ds(row0 + b * _BLK, _BLK), :],
            bufs[b % _NBUF], sems[b % _NBUF])

    handles = [start(b) for b in range(_NBUF)]

    riota = lax.iota(jnp.int32, 16)
    shift_idx = jnp.maximum(riota - 1, 0)

    pcopy = pltpu.async_copy(em_hbm.at[pl.ds(row0 - 1, 1), :], pbuf, sems[0])

    prev_last = jnp.int32(-1)
    for b in range(_NBLK):
        if b == 0:
            pcopy.wait()
        handles[b % _NBUF].wait()
        buf = bufs[b % _NBUF]
        cs = []
        for r0 in range(0, _BLK, _RIF):
            rows = [(buf, r0 + r) for r in range(_RIF)]
            if b == 0 and r0 == 0:
                rows.append((pbuf, 0))
                res = _scan_rows(rows, riota)
                prev_last = res[_RIF]
                cs.extend(res[:_RIF])
            else:
                cs.extend(_scan_rows(rows, riota))
        idxv = jnp.full((16,), -1, jnp.int32)
        for r, c in enumerate(cs):
            idxv = jnp.where(riota == r, c, idxv)
        shifted = _take16(idxv, shift_idx)
        prevv = jnp.where(riota == 0, prev_last, shifted)
        keep = (idxv != prevv) & (idxv != BLANK)
        outv[pl.ds(b * _BLK, _BLK)] = jnp.where(keep, idxv, -1)
        prev_last = cs[_BLK - 1]
        if b + _NBUF < _NBLK:
            handles[b % _NBUF] = start(b + _NBUF)

    pltpu.sync_copy(outv, out_hbm.at[pl.ds(wid * _ROWS_PER_W, _ROWS_PER_W)])


_sc_decode = pl.kernel(
    _sc_body,
    out_type=jax.ShapeDtypeStruct((_SC_ROWS,), jnp.int32),
    mesh=plsc.VectorSubcoreMesh(core_axis_name="c", subcore_axis_name="s"),
    compiler_params=pltpu.CompilerParams(
        needs_layout_passes=False, disable_bounds_checks=True),
    scratch_types=[
        pltpu.VMEM((_BLK, NUM_CLS), jnp.float32),
        pltpu.VMEM((_BLK, NUM_CLS), jnp.float32),
        pltpu.VMEM((_BLK, NUM_CLS), jnp.float32),
        pltpu.VMEM((_BLK, NUM_CLS), jnp.float32),
        pltpu.VMEM((1, NUM_CLS), jnp.float32),
        pltpu.VMEM((_ROWS_PER_W,), jnp.int32),
        pltpu.SemaphoreType.DMA,
        pltpu.SemaphoreType.DMA,
        pltpu.SemaphoreType.DMA,
        pltpu.SemaphoreType.DMA,
    ],
)


def _tc_body(em_ref, out_ref, prev_ref):
    @pl.when(pl.program_id(0) == 0)
    def _init():
        prev_ref[0] = jnp.int32(-1)

    x = em_ref[...]
    m = jnp.max(x, axis=1, keepdims=True)
    colio = lax.broadcasted_iota(jnp.int32, x.shape, 1)
    cand = jnp.where(x == m, colio, _BIG)
    idx = jnp.min(cand, axis=1)
    ps = prev_ref[0]
    shifted = jnp.concatenate([jnp.full((1,), ps, jnp.int32), idx[:-1]])
    keep = (idx != shifted) & (idx != BLANK)
    out_ref[...] = jnp.where(keep, idx, -1)
    prev_ref[0] = idx[_TC_R - 1]


_tc_decode = pl.pallas_call(
    _tc_body,
    grid=(_TC_ROWS // _TC_R,),
    in_specs=[pl.BlockSpec((_TC_R, NUM_CLS), lambda i: (i, 0))],
    out_specs=pl.BlockSpec((_TC_R,), lambda i: (i,)),
    out_shape=jax.ShapeDtypeStruct((_TC_ROWS,), jnp.int32),
    scratch_shapes=[pltpu.SMEM((1,), jnp.int32)],
)


@jax.jit
def kernel(emission):
    sc_out = _sc_decode(emission)
    tc_out = _tc_decode(emission)
    return jnp.concatenate([tc_out, sc_out])
